# baseline (speedup 1.0000x reference)
import functools

import jax
import jax.numpy as jnp
from jax import lax
from jax.experimental import pallas as pl
from jax.experimental.pallas import tpu as pltpu
from jax.experimental.pallas import tpu_sc as plsc

_T = 256
_WINDOW = 4096


def _tc_body(xb_ref, xsq_ref, cbt_ref, csq_ref, idx_ref, loss_ref):
    t = xb_ref.shape[0]
    k = cbt_ref.shape[1]
    xb = xb_ref[...]
    xsq = xsq_ref[...]
    chunk = 4096

    def window_min(a, b):
        wv, wi = None, None
        for c in range(a, b, chunk):
            mmc = jax.lax.dot_general(
                xb, cbt_ref[:, c:c + chunk], (((1,), (0,)), ((), ())),
                preferred_element_type=jnp.float32)
            dc = (xsq - mmc) + csq_ref[:, c:c + chunk]
            cv = jnp.min(dc, axis=1, keepdims=True)
            iot = jax.lax.broadcasted_iota(jnp.int32, (t, chunk), 1) + c
            ci = jnp.min(jnp.where(dc == cv, iot, k), axis=1)
            cv = cv[:, 0]
            if wv is None:
                wv, wi = cv, ci
            else:
                upd = (cv < wv) | ((cv == wv) & (ci < wi))
                wi = jnp.where(upd, ci, wi)
                wv = jnp.where(upd, cv, wv)
        return wv, wi

    acc_v = None
    acc_i = None
    acc_ex = None
    for a in range(0, k, _WINDOW):
        mv, wi = window_min(a, min(a + _WINDOW, k))
        mv_bf = mv.astype(jnp.bfloat16).astype(jnp.float32)
        if acc_v is None:
            acc_v, acc_i, acc_ex = mv_bf, wi, mv
        else:
            upd = mv < acc_v
            acc_i = jnp.where(upd, wi, acc_i)
            acc_ex = jnp.where(upd, mv, acc_ex)
            acc_v = jnp.where(upd, mv_bf, acc_v)

    idx_ref[0, 0, :] = acc_i

    @pl.when(pl.program_id(0) == 0)
    def _():
        loss_ref[...] = jnp.zeros_like(loss_ref)

    loss_ref[...] += jnp.sum(acc_ex).reshape(1, 1)


def _argmin_call(xb, xsq, cbt, csq):
    n, d = xb.shape
    k = cbt.shape[1]
    return pl.pallas_call(
        _tc_body,
        grid=(n // _T,),
        in_specs=[
            pl.BlockSpec((_T, d), lambda i: (i, 0)),
            pl.BlockSpec((_T, 1), lambda i: (i, 0)),
            pl.BlockSpec((d, k), lambda i: (0, 0)),
            pl.BlockSpec((1, k), lambda i: (0, 0)),
        ],
        out_specs=[
            pl.BlockSpec((1, 1, _T), lambda i: (i, 0, 0)),
            pl.BlockSpec((1, 1), lambda i: (0, 0)),
        ],
        out_shape=[
            jax.ShapeDtypeStruct((n // _T, 1, _T), jnp.int32),
            jax.ShapeDtypeStruct((1, 1), jnp.float32),
        ],
    )(xb, xsq, cbt, csq)


def _make_sc_gather(dp, n):
    info = plsc.get_sparse_core_info()
    nc, ns = info.num_cores, info.num_subcores
    nw = nc * ns
    b_per_w = n // nw
    n_chunk = 6
    chunk = b_per_w // n_chunk
    assert chunk * n_chunk == b_per_w and chunk <= 128 and chunk % 8 == 0
    mesh = plsc.VectorSubcoreMesh(core_axis_name="c", subcore_axis_name="s")

    @functools.partial(
        pl.kernel,
        mesh=mesh,
        out_type=jax.ShapeDtypeStruct((n, dp), jnp.float32),
        scratch_types=[
            pltpu.VMEM((n_chunk, chunk), jnp.int32),
            pltpu.VMEM((b_per_w, dp), jnp.float32),
            pltpu.SemaphoreType.DMA,
        ],
    )
    def gather_k(idx_hbm, table_hbm, out_hbm, idx_v, rows_v, sem):
        wid = lax.axis_index("s") * nc + lax.axis_index("c")
        pltpu.sync_copy(idx_hbm.at[wid], idx_v)
        copies = [
            pltpu.async_copy(
                table_hbm.at[idx_v.at[j]],
                rows_v.at[pl.ds(j * chunk, chunk)],
                sem,
            )
            for j in range(n_chunk)
        ]
        for cp in copies:
            cp.wait()
        pltpu.sync_copy(rows_v, out_hbm.at[pl.ds(wid * b_per_w, b_per_w)])

    return gather_k, nw, n_chunk, chunk


def kernel(x, codebook):
    b, s, d = x.shape
    n = b * s
    flat = x.reshape(n, d)
    xb = (2.0 * flat).astype(jnp.bfloat16)
    cbt = codebook.T.astype(jnp.bfloat16)
    xsq = jnp.sum(flat * flat, axis=1, keepdims=True)
    csq = jnp.sum(codebook * codebook, axis=1)[None, :]

    idx3, loss_sum = _argmin_call(xb, xsq, cbt, csq)
    idx_flat = idx3.reshape(n)

    dp = 128
    cb_pad = jnp.pad(codebook, ((0, 0), (0, dp - d)))
    gather_k, nw, n_chunk, chunk = _make_sc_gather(dp, n)
    quant = gather_k(idx_flat.reshape(nw, n_chunk, chunk), cb_pad)

    xq = quant[:, :d].reshape(b, s, d)
    indices = idx_flat.reshape(b, s)
    commit_loss = loss_sum[0, 0] / jnp.float32(n * d)
    return xq, indices, commit_loss


# speedup vs baseline: 1.0528x; 1.0528x over previous
import functools

import jax
import jax.numpy as jnp
from jax import lax
from jax.experimental import pallas as pl
from jax.experimental.pallas import tpu as pltpu
from jax.experimental.pallas import tpu_sc as plsc

_T = 1024
_WINDOW = 4096


def _tc_body(xb_ref, xsq_ref, cbt_ref, csq_ref, idx_ref, loss_ref):
    t = xb_ref.shape[0]
    k = cbt_ref.shape[1]
    xb = xb_ref[...]
    xsq = xsq_ref[...]
    chunk = 4096

    def window_min(a, b):
        wv, wi = None, None
        for c in range(a, b, chunk):
            mmc = jax.lax.dot_general(
                xb, cbt_ref[:, c:c + chunk], (((1,), (0,)), ((), ())),
                preferred_element_type=jnp.float32)
            dc = (xsq - mmc) + csq_ref[:, c:c + chunk]
            cv = jnp.min(dc, axis=1, keepdims=True)
            iot = jax.lax.broadcasted_iota(jnp.int32, (t, chunk), 1) + c
            ci = jnp.min(jnp.where(dc == cv, iot, k), axis=1)
            cv = cv[:, 0]
            if wv is None:
                wv, wi = cv, ci
            else:
                upd = (cv < wv) | ((cv == wv) & (ci < wi))
                wi = jnp.where(upd, ci, wi)
                wv = jnp.where(upd, cv, wv)
        return wv, wi

    acc_v = None
    acc_i = None
    acc_ex = None
    for a in range(0, k, _WINDOW):
        mv, wi = window_min(a, min(a + _WINDOW, k))
        mv_bf = mv.astype(jnp.bfloat16).astype(jnp.float32)
        if acc_v is None:
            acc_v, acc_i, acc_ex = mv_bf, wi, mv
        else:
            upd = mv < acc_v
            acc_i = jnp.where(upd, wi, acc_i)
            acc_ex = jnp.where(upd, mv, acc_ex)
            acc_v = jnp.where(upd, mv_bf, acc_v)

    idx_ref[0, 0, :] = acc_i

    @pl.when(pl.program_id(0) == 0)
    def _():
        loss_ref[...] = jnp.zeros_like(loss_ref)

    loss_ref[...] += jnp.sum(acc_ex).reshape(1, 1)


def _argmin_call(xb, xsq, cbt, csq):
    n, d = xb.shape
    k = cbt.shape[1]
    return pl.pallas_call(
        _tc_body,
        grid=(n // _T,),
        in_specs=[
            pl.BlockSpec((_T, d), lambda i: (i, 0)),
            pl.BlockSpec((_T, 1), lambda i: (i, 0)),
            pl.BlockSpec((d, k), lambda i: (0, 0)),
            pl.BlockSpec((1, k), lambda i: (0, 0)),
        ],
        out_specs=[
            pl.BlockSpec((1, 1, _T), lambda i: (i, 0, 0)),
            pl.BlockSpec((1, 1), lambda i: (0, 0)),
        ],
        out_shape=[
            jax.ShapeDtypeStruct((n // _T, 1, _T), jnp.int32),
            jax.ShapeDtypeStruct((1, 1), jnp.float32),
        ],
    )(xb, xsq, cbt, csq)


def _make_sc_gather(dp, n):
    info = plsc.get_sparse_core_info()
    nc, ns = info.num_cores, info.num_subcores
    nw = nc * ns
    b_per_w = n // nw
    n_chunk = 6
    chunk = b_per_w // n_chunk
    assert chunk * n_chunk == b_per_w and chunk <= 128 and chunk % 8 == 0
    mesh = plsc.VectorSubcoreMesh(core_axis_name="c", subcore_axis_name="s")

    @functools.partial(
        pl.kernel,
        mesh=mesh,
        out_type=jax.ShapeDtypeStruct((n, dp), jnp.float32),
        scratch_types=[
            pltpu.VMEM((n_chunk, chunk), jnp.int32),
            pltpu.VMEM((b_per_w, dp), jnp.float32),
            pltpu.SemaphoreType.DMA,
        ],
    )
    def gather_k(idx_hbm, table_hbm, out_hbm, idx_v, rows_v, sem):
        wid = lax.axis_index("s") * nc + lax.axis_index("c")
        pltpu.sync_copy(idx_hbm.at[wid], idx_v)
        copies = [
            pltpu.async_copy(
                table_hbm.at[idx_v.at[j]],
                rows_v.at[pl.ds(j * chunk, chunk)],
                sem,
            )
            for j in range(n_chunk)
        ]
        for cp in copies:
            cp.wait()
        pltpu.sync_copy(rows_v, out_hbm.at[pl.ds(wid * b_per_w, b_per_w)])

    return gather_k, nw, n_chunk, chunk


def kernel(x, codebook):
    b, s, d = x.shape
    n = b * s
    flat = x.reshape(n, d)
    xb = (2.0 * flat).astype(jnp.bfloat16)
    cbt = codebook.T.astype(jnp.bfloat16)
    xsq = jnp.sum(flat * flat, axis=1, keepdims=True)
    csq = jnp.sum(codebook * codebook, axis=1)[None, :]

    idx3, loss_sum = _argmin_call(xb, xsq, cbt, csq)
    idx_flat = idx3.reshape(n)

    dp = 128
    cb_pad = jnp.pad(codebook, ((0, 0), (0, dp - d)))
    gather_k, nw, n_chunk, chunk = _make_sc_gather(dp, n)
    quant = gather_k(idx_flat.reshape(nw, n_chunk, chunk), cb_pad)

    xq = quant[:, :d].reshape(b, s, d)
    indices = idx_flat.reshape(b, s)
    commit_loss = loss_sum[0, 0] / jnp.float32(n * d)
    return xq, indices, commit_loss
